# Initial kernel scaffold; baseline (speedup 1.0000x reference)
import jax
import jax.numpy as jnp
from jax.experimental import pallas as pl


def kernel(x, Wi, bi, Wg, bg, W1, b1, W2, b2, Wo, bo):
    raise NotImplementedError("write your pallas kernel here")


# speedup vs baseline: 1.2108x; 1.2108x over previous
import functools

import jax
import jax.numpy as jnp
from jax.experimental import pallas as pl
from jax.experimental.pallas import tpu as pltpu

T, DI, DH, DM, DO, E = 2048, 1024, 1024, 1024, 1024, 8
BT = 256
NT = T // BT
LANE = 128
_NEG = -1e30


def _proj_router_kern(x_ref, wi_ref, bi_ref, wg_ref, bg_ref, h_ref, comb_ref):
    h = jnp.dot(x_ref[...], wi_ref[...], preferred_element_type=jnp.float32)
    h = h + bi_ref[...]
    h_ref[...] = h
    logits = jnp.dot(h, wg_ref[...], preferred_element_type=jnp.float32)
    logits = logits + bg_ref[...]
    col = jax.lax.broadcasted_iota(jnp.int32, logits.shape, 1)
    logits = jnp.where(col < E, logits, _NEG)
    v1 = jnp.max(logits, axis=-1, keepdims=True)
    i1 = jnp.min(jnp.where(logits == v1, col, LANE), axis=-1, keepdims=True)
    l2 = jnp.where(col == i1, _NEG, logits)
    v2 = jnp.max(l2, axis=-1, keepdims=True)
    i2 = jnp.min(jnp.where(l2 == v2, col, LANE), axis=-1, keepdims=True)
    p1 = 1.0 / (1.0 + jnp.exp(v2 - v1))
    p2 = 1.0 - p1
    comb_ref[...] = jnp.where(col == i1, p1, 0.0) + jnp.where(col == i2, p2, 0.0)


def _experts_kern(h_ref, comb_ref, w1_ref, b1_ref, w2_ref, b2_ref, out_ref,
                  acc_ref):
    e = pl.program_id(0)
    t = pl.program_id(1)
    rows = pl.ds(t * BT, BT)
    h = h_ref[rows, :]
    h1 = jnp.maximum(
        jnp.dot(h, w1_ref[0], preferred_element_type=jnp.float32) + b1_ref[0],
        0.0)
    h2 = jnp.dot(h1, w2_ref[0], preferred_element_type=jnp.float32) + b2_ref[0]
    comb = comb_ref[rows, :]
    col = jax.lax.broadcasted_iota(jnp.int32, comb.shape, 1)
    w = jnp.sum(jnp.where(col == e, comb, 0.0), axis=-1, keepdims=True)
    contrib = w * h2

    @pl.when(e == 0)
    def _():
        acc_ref[rows, :] = contrib

    @pl.when(e > 0)
    def _():
        acc_ref[rows, :] = acc_ref[rows, :] + contrib

    @pl.when(e == E - 1)
    def _():
        out_ref[...] = acc_ref[rows, :]


def _outproj_kern(m_ref, wo_ref, bo_ref, out_ref):
    out_ref[...] = jnp.dot(
        m_ref[...], wo_ref[...], preferred_element_type=jnp.float32) + bo_ref[...]


def kernel(x, Wi, bi, Wg, bg, W1, b1, W2, b2, Wo, bo):
    wg_pad = jnp.zeros((DH, LANE), jnp.float32).at[:, :E].set(Wg)
    bg_row = jnp.zeros((1, LANE), jnp.float32).at[0, :E].set(bg)

    h, comb = pl.pallas_call(
        _proj_router_kern,
        grid=(NT,),
        in_specs=[
            pl.BlockSpec((BT, DI), lambda t: (t, 0)),
            pl.BlockSpec((DI, DH), lambda t: (0, 0)),
            pl.BlockSpec((1, DH), lambda t: (0, 0)),
            pl.BlockSpec((DH, LANE), lambda t: (0, 0)),
            pl.BlockSpec((1, LANE), lambda t: (0, 0)),
        ],
        out_specs=[
            pl.BlockSpec((BT, DH), lambda t: (t, 0)),
            pl.BlockSpec((BT, LANE), lambda t: (t, 0)),
        ],
        out_shape=[
            jax.ShapeDtypeStruct((T, DH), jnp.float32),
            jax.ShapeDtypeStruct((T, LANE), jnp.float32),
        ],
    )(x, Wi, bi.reshape(1, DH), wg_pad, bg_row)

    moe = pl.pallas_call(
        _experts_kern,
        grid=(E, NT),
        in_specs=[
            pl.BlockSpec((T, DH), lambda e, t: (0, 0)),
            pl.BlockSpec((T, LANE), lambda e, t: (0, 0)),
            pl.BlockSpec((1, DH, DM), lambda e, t: (e, 0, 0)),
            pl.BlockSpec((1, 1, DM), lambda e, t: (e, 0, 0)),
            pl.BlockSpec((1, DM, DH), lambda e, t: (e, 0, 0)),
            pl.BlockSpec((1, 1, DH), lambda e, t: (e, 0, 0)),
        ],
        out_specs=pl.BlockSpec((BT, DH), lambda e, t: (t, 0)),
        out_shape=jax.ShapeDtypeStruct((T, DH), jnp.float32),
        scratch_shapes=[pltpu.VMEM((T, DH), jnp.float32)],
    )(h, comb, W1, b1.reshape(E, 1, DM), W2, b2.reshape(E, 1, DH))

    out = pl.pallas_call(
        _outproj_kern,
        grid=(NT,),
        in_specs=[
            pl.BlockSpec((BT, DH), lambda t: (t, 0)),
            pl.BlockSpec((DH, DO), lambda t: (0, 0)),
            pl.BlockSpec((1, DO), lambda t: (0, 0)),
        ],
        out_specs=pl.BlockSpec((BT, DO), lambda t: (t, 0)),
        out_shape=jax.ShapeDtypeStruct((T, DO), jnp.float32),
    )(moe, Wo, bo.reshape(1, DO))
    return out
